# baseline (device time: 141994 ns/iter reference)
import jax
import jax.numpy as jnp
from jax import lax
from jax.experimental import pallas as pl
from jax.experimental.pallas import tpu as pltpu

K = 16


def kernel(x):
    m_per, n = x.shape
    half = m_per // 2
    rows = half // K

    def body(x_ref, out_ref, x_send_sems, x_recv_sems, y_send_sems, y_recv_sems,
             local_sem):
        mx = lax.axis_index("x")
        my = lax.axis_index("y")
        peer_x = (1 - mx, my)
        peer_y = (mx, 1 - my)

        barrier_sem = pltpu.get_barrier_semaphore()
        for p in (peer_x, peer_y):
            pl.semaphore_signal(
                barrier_sem, inc=1, device_id=p,
                device_id_type=pl.DeviceIdType.MESH,
            )
        pl.semaphore_wait(barrier_sem, 2)

        x_rdmas = []
        for k in range(K):
            src_off = my * half + k * rows
            dst_off = mx * m_per + my * half + k * rows
            r = pltpu.make_async_remote_copy(
                src_ref=x_ref.at[pl.ds(src_off, rows), :],
                dst_ref=out_ref.at[pl.ds(dst_off, rows), :],
                send_sem=x_send_sems.at[k],
                recv_sem=x_recv_sems.at[k],
                device_id=peer_x,
                device_id_type=pl.DeviceIdType.MESH,
            )
            r.start()
            x_rdmas.append(r)

        local_copy = pltpu.make_async_copy(
            x_ref, out_ref.at[pl.ds(mx * m_per, m_per), :], local_sem,
        )
        local_copy.start()

        y_rdmas = []
        for k in range(K):
            in_off = (1 - mx) * m_per + my * half + k * rows
            x_recv = pltpu.make_async_remote_copy(
                src_ref=x_ref.at[pl.ds(0, rows), :],
                dst_ref=out_ref.at[pl.ds(in_off, rows), :],
                send_sem=x_send_sems.at[k],
                recv_sem=x_recv_sems.at[k],
                device_id=peer_x,
                device_id_type=pl.DeviceIdType.MESH,
            )
            x_recv.wait_recv()
            r = pltpu.make_async_remote_copy(
                src_ref=out_ref.at[pl.ds(in_off, rows), :],
                dst_ref=out_ref.at[pl.ds(in_off, rows), :],
                send_sem=y_send_sems.at[k],
                recv_sem=y_recv_sems.at[k],
                device_id=peer_y,
                device_id_type=pl.DeviceIdType.MESH,
            )
            r.start()
            y_rdmas.append(r)

        for k in range(K):
            in_off = (1 - mx) * m_per + (1 - my) * half + k * rows
            y_recv = pltpu.make_async_remote_copy(
                src_ref=x_ref.at[pl.ds(0, rows), :],
                dst_ref=out_ref.at[pl.ds(in_off, rows), :],
                send_sem=y_send_sems.at[k],
                recv_sem=y_recv_sems.at[k],
                device_id=peer_y,
                device_id_type=pl.DeviceIdType.MESH,
            )
            y_recv.wait_recv()

        for r in x_rdmas:
            r.wait_send()
        for r in y_rdmas:
            r.wait_send()
        local_copy.wait()

    return pl.pallas_call(
        body,
        out_shape=jax.ShapeDtypeStruct((2 * m_per, n), x.dtype),
        in_specs=[pl.BlockSpec(memory_space=pltpu.VMEM)],
        out_specs=pl.BlockSpec(memory_space=pltpu.VMEM),
        scratch_shapes=[
            pltpu.SemaphoreType.DMA((K,)),
            pltpu.SemaphoreType.DMA((K,)),
            pltpu.SemaphoreType.DMA((K,)),
            pltpu.SemaphoreType.DMA((K,)),
            pltpu.SemaphoreType.DMA,
        ],
        compiler_params=pltpu.CompilerParams(collective_id=0),
    )(x)


# device time: 130242 ns/iter; 1.0902x vs baseline; 1.0902x over previous
import jax
import jax.numpy as jnp
from jax import lax
from jax.experimental import pallas as pl
from jax.experimental.pallas import tpu as pltpu

K = 32


def kernel(x):
    m_per, n = x.shape
    half = m_per // 2
    rows = half // K

    def body(x_ref, out_ref, x_send_sems, x_recv_sems, y_send_sems, y_recv_sems,
             local_sem):
        mx = lax.axis_index("x")
        my = lax.axis_index("y")
        peer_x = (1 - mx, my)
        peer_y = (mx, 1 - my)

        barrier_sem = pltpu.get_barrier_semaphore()
        for p in (peer_x, peer_y):
            pl.semaphore_signal(
                barrier_sem, inc=1, device_id=p,
                device_id_type=pl.DeviceIdType.MESH,
            )
        pl.semaphore_wait(barrier_sem, 2)

        x_rdmas = []
        for k in range(K):
            src_off = my * half + k * rows
            dst_off = mx * m_per + my * half + k * rows
            r = pltpu.make_async_remote_copy(
                src_ref=x_ref.at[pl.ds(src_off, rows), :],
                dst_ref=out_ref.at[pl.ds(dst_off, rows), :],
                send_sem=x_send_sems.at[k],
                recv_sem=x_recv_sems.at[k],
                device_id=peer_x,
                device_id_type=pl.DeviceIdType.MESH,
            )
            r.start()
            x_rdmas.append(r)

        local_copy = pltpu.make_async_copy(
            x_ref, out_ref.at[pl.ds(mx * m_per, m_per), :], local_sem,
        )
        local_copy.start()

        y_rdmas = []
        for k in range(K):
            in_off = (1 - mx) * m_per + my * half + k * rows
            x_recv = pltpu.make_async_remote_copy(
                src_ref=x_ref.at[pl.ds(0, rows), :],
                dst_ref=out_ref.at[pl.ds(in_off, rows), :],
                send_sem=x_send_sems.at[k],
                recv_sem=x_recv_sems.at[k],
                device_id=peer_x,
                device_id_type=pl.DeviceIdType.MESH,
            )
            x_recv.wait_recv()
            r = pltpu.make_async_remote_copy(
                src_ref=out_ref.at[pl.ds(in_off, rows), :],
                dst_ref=out_ref.at[pl.ds(in_off, rows), :],
                send_sem=y_send_sems.at[k],
                recv_sem=y_recv_sems.at[k],
                device_id=peer_y,
                device_id_type=pl.DeviceIdType.MESH,
            )
            r.start()
            y_rdmas.append(r)

        for k in range(K):
            in_off = (1 - mx) * m_per + (1 - my) * half + k * rows
            y_recv = pltpu.make_async_remote_copy(
                src_ref=x_ref.at[pl.ds(0, rows), :],
                dst_ref=out_ref.at[pl.ds(in_off, rows), :],
                send_sem=y_send_sems.at[k],
                recv_sem=y_recv_sems.at[k],
                device_id=peer_y,
                device_id_type=pl.DeviceIdType.MESH,
            )
            y_recv.wait_recv()

        for r in x_rdmas:
            r.wait_send()
        for r in y_rdmas:
            r.wait_send()
        local_copy.wait()

    return pl.pallas_call(
        body,
        out_shape=jax.ShapeDtypeStruct((2 * m_per, n), x.dtype),
        in_specs=[pl.BlockSpec(memory_space=pl.ANY)],
        out_specs=pl.BlockSpec(memory_space=pl.ANY),
        scratch_shapes=[
            pltpu.SemaphoreType.DMA((K,)),
            pltpu.SemaphoreType.DMA((K,)),
            pltpu.SemaphoreType.DMA((K,)),
            pltpu.SemaphoreType.DMA((K,)),
            pltpu.SemaphoreType.DMA,
        ],
        compiler_params=pltpu.CompilerParams(collective_id=0),
    )(x)
